# Initial kernel scaffold; baseline (speedup 1.0000x reference)
import jax
import jax.numpy as jnp
from jax.experimental import pallas as pl


def kernel(seq1, edge_index, edge_weight, sparse, W_gcn, b_gcn, W_mlp, b_mlp):
    raise NotImplementedError("write your pallas kernel here")


# speedup vs baseline: 3.1550x; 3.1550x over previous
import functools

import jax
import jax.numpy as jnp
from jax import lax
from jax.experimental import pallas as pl
from jax.experimental.pallas import tpu as pltpu
from jax.experimental.pallas import tpu_sc as plsc

N = 10000
E = 320000
F = 128
NC = 2
NS = 16
NW = NC * NS
CH = 128
KC = 80
KB = 8
NB = KC // KB
EW = KC * CH
EPAD = NW * EW
NP = 10240
ZR = NP // NS


def _mm_body(x_ref, w_ref, o_ref):
    o_ref[...] = jnp.dot(x_ref[...], w_ref[...],
                         preferred_element_type=jnp.float32)


def _tc_mm(x, w):
    r = 1000
    return pl.pallas_call(
        _mm_body,
        grid=(N // r,),
        in_specs=[
            pl.BlockSpec((r, F), lambda i: (i, 0)),
            pl.BlockSpec((F, F), lambda i: (0, 0)),
        ],
        out_specs=pl.BlockSpec((r, F), lambda i: (i, 0)),
        out_shape=jax.ShapeDtypeStruct((N, F), jnp.float32),
    )(x, w)


_mesh = plsc.VectorSubcoreMesh(core_axis_name="c", subcore_axis_name="s",
                               num_cores=NC, num_subcores=NS)


@functools.partial(
    pl.kernel,
    out_type=jax.ShapeDtypeStruct((NC, NP, F), jnp.float32),
    mesh=_mesh,
    scratch_types=[
        pltpu.VMEM((KB, CH), jnp.int32),
        pltpu.VMEM((KB, CH), jnp.int32),
        pltpu.VMEM((KB, CH), jnp.float32),
        pltpu.VMEM((2, CH, F), jnp.float32),
        pltpu.VMEM_SHARED((NP, F), jnp.float32),
        pltpu.SemaphoreType.DMA,
        pltpu.SemaphoreType.DMA,
    ],
)
def _sc_agg(h_hbm, src_hbm, dst_hbm, w_hbm, out_hbm,
            src_v, dst_v, w_v, rows_v, acc, gsem, ssem):
    c = lax.axis_index("c")
    s = lax.axis_index("s")
    wid = c * NS + s

    def _zero_row(r, carry):
        for k in range(F // 16):
            rows_v[0, r, pl.ds(k * 16, 16)] = jnp.zeros((16,), jnp.float32)
        return carry
    lax.fori_loop(0, CH, _zero_row, 0)

    base = s * ZR
    for i in range(ZR // CH):
        pltpu.sync_copy(rows_v.at[0], acc.at[pl.ds(base + i * CH, CH)])
    plsc.subcore_barrier()

    def _block(b, carry):
        pltpu.sync_copy(src_hbm.at[wid, pl.ds(b * KB, KB)], src_v)
        pltpu.sync_copy(dst_hbm.at[wid, pl.ds(b * KB, KB)], dst_v)
        pltpu.sync_copy(w_hbm.at[wid, pl.ds(b * KB, KB)], w_v)

        def _chunk(j, carry1):
            pltpu.async_copy(h_hbm.at[src_v.at[j]], rows_v.at[0], gsem).wait()

            def _group(g, carry2):
                wv = w_v[j, pl.ds(g * 16, 16)]
                for l in range(16):
                    wl = jnp.full((16,), wv[l], jnp.float32)
                    e = g * 16 + l
                    for k in range(F // 16):
                        sl = pl.ds(k * 16, 16)
                        rows_v[0, e, sl] = rows_v[0, e, sl] * wl
                return carry2
            lax.fori_loop(0, CH // 16, _group, 0)

            pltpu.async_copy(rows_v.at[0], acc.at[dst_v.at[j]], ssem,
                             add=True).wait()
            return carry1
        lax.fori_loop(0, KB, _chunk, 0)
        return carry
    lax.fori_loop(0, NB, _block, 0)

    plsc.subcore_barrier()
    for i in range(ZR // CH):
        pltpu.sync_copy(acc.at[pl.ds(base + i * CH, CH)],
                        out_hbm.at[c, pl.ds(base + i * CH, CH)])


def _post_body(p_ref, bg_ref, wm_ref, bm_ref, h1_ref, pr_ref):
    agg = p_ref[0] + p_ref[1]
    h1 = jnp.maximum(agg + bg_ref[...], 0.0)
    h1_ref[...] = h1
    pr_ref[...] = jnp.dot(h1, wm_ref[...],
                          preferred_element_type=jnp.float32) + bm_ref[...]


def _tc_post(p, bg, wm, bm):
    r = 1000
    return pl.pallas_call(
        _post_body,
        grid=(N // r,),
        in_specs=[
            pl.BlockSpec((NC, r, F), lambda i: (0, i, 0)),
            pl.BlockSpec((1, F), lambda i: (0, 0)),
            pl.BlockSpec((F, F), lambda i: (0, 0)),
            pl.BlockSpec((1, F), lambda i: (0, 0)),
        ],
        out_specs=[
            pl.BlockSpec((r, F), lambda i: (i, 0)),
            pl.BlockSpec((r, F), lambda i: (i, 0)),
        ],
        out_shape=[
            jax.ShapeDtypeStruct((N, F), jnp.float32),
            jax.ShapeDtypeStruct((N, F), jnp.float32),
        ],
    )(p, bg, wm, bm)


def kernel(seq1, edge_index, edge_weight, sparse, W_gcn, b_gcn, W_mlp, b_mlp):
    x = seq1[0]
    h = _tc_mm(x, W_gcn)

    src = edge_index[0].astype(jnp.int32)
    dst = edge_index[1].astype(jnp.int32)
    pad = EPAD - E
    zpad_i = jnp.zeros((pad,), jnp.int32)
    src_p = jnp.concatenate([src, zpad_i]).reshape(NW, KC, CH)
    dst_p = jnp.concatenate([dst, zpad_i]).reshape(NW, KC, CH)
    w_p = jnp.concatenate([edge_weight.astype(jnp.float32),
                           jnp.zeros((pad,), jnp.float32)]).reshape(NW, KC, CH)

    partials = _sc_agg(h, src_p, dst_p, w_p)

    h1, preds = _tc_post(partials, b_gcn.reshape(1, F), W_mlp,
                         b_mlp.reshape(1, F))
    return (h1[None], preds[None])
